# baseline (speedup 1.0000x reference)
import functools

import jax
import jax.numpy as jnp
from jax.experimental import pallas as pl
from jax.experimental.pallas import tpu as pltpu


def _fold(q):
    return (q[:, 0:128] + q[:, 128:256]) + (q[:, 256:384] + q[:, 384:512])


def _body(skls_ref, masks_ref, gt_ref, out_ref, acc_ref, *, B, C, H, W, R):
    b = pl.program_id(0)
    rb = pl.program_id(1)
    nrb = H // R

    @pl.when((b == 0) & (rb == 0))
    def _init():
        acc_ref[...] = jnp.zeros_like(acc_ref)

    x_min = skls_ref[b, 0, 0]
    x_max = skls_ref[b, 0, 0]
    y_min = skls_ref[b, 0, 1]
    y_max = skls_ref[b, 0, 1]
    for j in range(1, 17):
        x_min = jnp.minimum(x_min, skls_ref[b, j, 0])
        x_max = jnp.maximum(x_max, skls_ref[b, j, 0])
        y_min = jnp.minimum(y_min, skls_ref[b, j, 1])
        y_max = jnp.maximum(y_max, skls_ref[b, j, 1])
    x_min = jnp.maximum(x_min.astype(jnp.int32) - 10, 0)
    x_max = jnp.minimum(x_max.astype(jnp.int32) + 10, W)
    y_min = jnp.maximum(y_min.astype(jnp.int32) - 10, 0)
    y_max = jnp.minimum(y_max.astype(jnp.int32) + 10, H)

    cols = jax.lax.broadcasted_iota(jnp.int32, (8, 128), 1)
    row_iota = jax.lax.broadcasted_iota(jnp.int32, (8, 128), 0)

    zeros = jnp.zeros((8, 128), jnp.float32)
    accs = [zeros] * (4 * C)
    base = rb * R
    colms = [(cols >= x_min - w * 128) & (cols < x_max - w * 128)
             for w in range(W // 128)]
    one = jnp.ones((8, 128), jnp.float32)
    for c in range(C):
        def slice_body(s, carry, c=c):
            a0, a1, a2, a3 = carry
            r0 = s * 8
            y_lo = y_min - (base + r0)
            y_hi = y_max - (base + r0)
            rowm = (row_iota >= y_lo) & (row_iota < y_hi)
            for w in range(W // 128):
                box = rowm & colms[w]
                boxf = jnp.where(box, 1.0, 0.0)
                gt = gt_ref[0, pl.ds(r0, 8), w * 128:(w + 1) * 128]
                x = masks_ref[0, c, pl.ds(r0, 8), w * 128:(w + 1) * 128]
                e = jnp.exp2(jnp.abs(x) * jnp.float32(-1.4426950408889634))
                sp = jnp.maximum(x, 0.0) + jnp.log(1.0 + e)
                bce1 = sp - x
                posf = jnp.where(gt == (c + 1), 1.0, 0.0)
                negf = jnp.abs(boxf - posf)
                a0 = a0 + bce1 * posf
                a1 = a1 + posf
                a2 = a2 + sp * negf
                a3 = a3 + negf
            return a0, a1, a2, a3
        a0, a1, a2, a3 = jax.lax.fori_loop(
            0, R // 8, slice_body, (zeros, zeros, zeros, zeros), unroll=4)
        accs[4 * c + 0] = a0
        accs[4 * c + 1] = a1
        accs[4 * c + 2] = a2
        accs[4 * c + 3] = a3

    for q in range(4 * C):
        acc_ref[q] += accs[q]

    @pl.when((b == B - 1) & (rb == nrb - 1))
    def _fin():
        loss = 0.0
        for c in range(C):
            loss += 0.1 * jnp.sum(acc_ref[4 * c + 0]) / jnp.sum(acc_ref[4 * c + 1])
            loss += 0.9 * jnp.sum(acc_ref[4 * c + 2]) / jnp.sum(acc_ref[4 * c + 3])
        out_ref[0] = loss


def kernel(skls, masks, gt_masks):
    B, C, H, W = masks.shape
    R = 512
    grid = (B, H // R) if R < H else (B, 1)
    out = pl.pallas_call(
        functools.partial(_body, B=B, C=C, H=H, W=W, R=R),
        grid=grid,
        in_specs=[
            pl.BlockSpec(memory_space=pltpu.SMEM),
            pl.BlockSpec((1, C, R, W), lambda b, r: (b, 0, r, 0)),
            pl.BlockSpec((1, R, W), lambda b, r: (b, r, 0)),
        ],
        out_specs=pl.BlockSpec(memory_space=pltpu.SMEM),
        out_shape=jax.ShapeDtypeStruct((1,), masks.dtype),
        scratch_shapes=[pltpu.VMEM((4 * C, 8, 128), jnp.float32)],
    )(skls, masks, gt_masks)
    return out[0]


# speedup vs baseline: 1.2531x; 1.2531x over previous
import functools

import jax
import jax.numpy as jnp
from jax.experimental import pallas as pl
from jax.experimental.pallas import tpu as pltpu


def _fold(q):
    return (q[:, 0:128] + q[:, 128:256]) + (q[:, 256:384] + q[:, 384:512])


def _body(skls_ref, masks_ref, gt_ref, out_ref, acc_ref, *, B, C, H, W, R):
    b = pl.program_id(0)
    rb = pl.program_id(1)
    nrb = H // R

    @pl.when((b == 0) & (rb == 0))
    def _init():
        acc_ref[...] = jnp.zeros_like(acc_ref)

    x_min = skls_ref[b, 0, 0]
    x_max = skls_ref[b, 0, 0]
    y_min = skls_ref[b, 0, 1]
    y_max = skls_ref[b, 0, 1]
    for j in range(1, 17):
        x_min = jnp.minimum(x_min, skls_ref[b, j, 0])
        x_max = jnp.maximum(x_max, skls_ref[b, j, 0])
        y_min = jnp.minimum(y_min, skls_ref[b, j, 1])
        y_max = jnp.maximum(y_max, skls_ref[b, j, 1])
    x_min = jnp.maximum(x_min.astype(jnp.int32) - 10, 0)
    x_max = jnp.minimum(x_max.astype(jnp.int32) + 10, W)
    y_min = jnp.maximum(y_min.astype(jnp.int32) - 10, 0)
    y_max = jnp.minimum(y_max.astype(jnp.int32) + 10, H)

    cols = jax.lax.broadcasted_iota(jnp.int32, (8, 128), 1)
    row_iota = jax.lax.broadcasted_iota(jnp.int32, (8, 128), 0)

    zeros = jnp.zeros((8, 128), jnp.float32)
    accs = [zeros] * (4 * C)
    base = rb * R
    colfs = [jnp.where((cols >= x_min - w * 128) & (cols < x_max - w * 128),
                       1.0, 0.0) for w in range(W // 128)]
    one = jnp.ones((8, 128), jnp.float32)
    for c in range(C):
        a0, a1, a2, a3 = zeros, zeros, zeros, zeros
        for s in range(R // 8):
            r0 = s * 8
            y_lo = y_min - (base + r0)
            y_hi = y_max - (base + r0)
            rowf = jnp.where((row_iota >= y_lo) & (row_iota < y_hi), 1.0, 0.0)
            for w in range(W // 128):
                boxf = rowf * colfs[w]
                gt = gt_ref[0, r0:r0 + 8, w * 128:(w + 1) * 128]
                x = masks_ref[0, c, r0:r0 + 8, w * 128:(w + 1) * 128]
                e = jnp.exp2(jnp.abs(x) * jnp.float32(-1.4426950408889634))
                sp = jnp.maximum(x, 0.0) + jnp.log(1.0 + e)
                bce1 = sp - x
                posf = jnp.where(gt == (c + 1), 1.0, 0.0)
                negf = jnp.abs(boxf - posf)
                a0 = a0 + bce1 * posf
                a1 = a1 + posf
                a2 = a2 + sp * negf
                a3 = a3 + negf
        accs[4 * c + 0] = a0
        accs[4 * c + 1] = a1
        accs[4 * c + 2] = a2
        accs[4 * c + 3] = a3

    for q in range(4 * C):
        acc_ref[q] += accs[q]

    @pl.when((b == B - 1) & (rb == nrb - 1))
    def _fin():
        loss = 0.0
        for c in range(C):
            loss += 0.1 * jnp.sum(acc_ref[4 * c + 0]) / jnp.sum(acc_ref[4 * c + 1])
            loss += 0.9 * jnp.sum(acc_ref[4 * c + 2]) / jnp.sum(acc_ref[4 * c + 3])
        out_ref[0] = loss


def kernel(skls, masks, gt_masks):
    B, C, H, W = masks.shape
    R = 512
    grid = (B, H // R) if R < H else (B, 1)
    out = pl.pallas_call(
        functools.partial(_body, B=B, C=C, H=H, W=W, R=R),
        grid=grid,
        in_specs=[
            pl.BlockSpec(memory_space=pltpu.SMEM),
            pl.BlockSpec((1, C, R, W), lambda b, r: (b, 0, r, 0)),
            pl.BlockSpec((1, R, W), lambda b, r: (b, r, 0)),
        ],
        out_specs=pl.BlockSpec(memory_space=pltpu.SMEM),
        out_shape=jax.ShapeDtypeStruct((1,), masks.dtype),
        scratch_shapes=[pltpu.VMEM((4 * C, 8, 128), jnp.float32)],
    )(skls, masks, gt_masks)
    return out[0]
